# baseline (speedup 1.0000x reference)
import functools

import jax
import jax.numpy as jnp
from jax import lax
from jax.experimental import pallas as pl
from jax.experimental.pallas import tpu as pltpu
from jax.experimental.pallas import tpu_sc as plsc

BATCH = 4096
SEQ = 200
DIMS = 64
LANES = 16
NW = 32
B_PER_W = BATCH // NW
SB = 2
NBUF = 2
NBLOCKS = SEQ // SB
ROWS_PER_BLOCK = SB * B_PER_W


def _suffix_embed_kernel(x_hbm, pe_hbm, out_hbm, x_v, idx_v, pe_v,
                         buf0, buf1, wsem0, wsem1):
    wid = lax.axis_index("s") * 2 + lax.axis_index("c")
    b_base = wid * B_PER_W

    pltpu.sync_copy(x_hbm.at[pl.ds(b_base * SEQ, B_PER_W * SEQ)], x_v)
    pltpu.sync_copy(pe_hbm, pe_v)

    iota = lax.iota(jnp.int32, LANES)
    one = jnp.ones((LANES,), jnp.int32)
    zero = jnp.zeros((LANES,), jnp.int32)

    row_bases = [(g * LANES + iota) * SEQ for g in range(B_PER_W // LANES)]

    def scan_body(i, accs):
        s = SEQ - 1 - i
        out_accs = []
        for g, acc in enumerate(accs):
            col = plsc.load_gather(x_v, [row_bases[g] + s])
            acc = acc + jnp.where(col != 0, one, zero)
            idx_v[pl.ds(s * B_PER_W + g * LANES, LANES)] = acc
            out_accs.append(acc)
        return tuple(out_accs)

    lax.fori_loop(0, SEQ, scan_body, tuple(zero for _ in range(B_PER_W // LANES)))

    bufs = (buf0, buf1)
    wsems = (wsem0, wsem1)
    d_iotas = [iota + k * LANES for k in range(DIMS // LANES)]

    def write_block(i, b, wait):
        for k in range(SB):
            cp = pltpu.make_async_copy(
                bufs[b].at[pl.ds(k * B_PER_W, B_PER_W), :],
                out_hbm.at[pl.ds((i * SB + k) * BATCH + b_base, B_PER_W), :],
                wsems[b],
            )
            if wait:
                cp.wait()
            else:
                cp.start()

    def fill_block(i, b):
        buf = bufs[b]

        def grp_body(g16, carry, buf=buf, i=i):
            base = g16 * LANES
            T = idx_v[pl.ds(i * ROWS_PER_BLOCK + base, LANES)] * DIMS
            for q in range(LANES // 8):
                gs = []
                for l in range(q * 8, q * 8 + 8):
                    t64 = T[l]
                    gs.append([plsc.load_gather(pe_v, [t64 + d_iotas[k]])
                               for k in range(DIMS // LANES)])
                for li, l in enumerate(range(q * 8, q * 8 + 8)):
                    for k in range(DIMS // LANES):
                        buf[base + l, pl.ds(k * LANES, LANES)] = gs[li][k]
            return carry

        lax.fori_loop(0, ROWS_PER_BLOCK // LANES, grp_body, 0)

    def outer(o, carry):
        j = o * NBUF
        for b in range(NBUF):
            i = j + b

            @pl.when(o > 0)
            def _(b=b, i=i):
                write_block(i - NBUF, b, wait=True)

            fill_block(i, b)
            write_block(i, b, wait=False)
        return carry

    lax.fori_loop(0, NBLOCKS // NBUF, outer, 0)

    for b in range(NBUF):
        write_block(NBLOCKS - NBUF + b, b, wait=True)


@jax.jit
def kernel(x, pe):
    mesh = plsc.VectorSubcoreMesh(core_axis_name="c", subcore_axis_name="s")
    fn = functools.partial(
        pl.kernel,
        mesh=mesh,
        compiler_params=pltpu.CompilerParams(
            needs_layout_passes=False, use_tc_tiling_on_sc=False
        ),
        out_type=jax.ShapeDtypeStruct((SEQ * BATCH, DIMS), jnp.float32),
        scratch_types=[
            pltpu.VMEM((B_PER_W * SEQ,), jnp.int32),
            pltpu.VMEM((SEQ * B_PER_W,), jnp.int32),
            pltpu.VMEM((256 * DIMS,), jnp.float32),
            pltpu.VMEM((ROWS_PER_BLOCK, DIMS), jnp.float32),
            pltpu.VMEM((ROWS_PER_BLOCK, DIMS), jnp.float32),
            pltpu.SemaphoreType.DMA,
            pltpu.SemaphoreType.DMA,
        ],
    )(_suffix_embed_kernel)
    return fn(x.reshape(-1), pe.reshape(-1)).reshape(SEQ, BATCH, DIMS)


# speedup vs baseline: 1.0059x; 1.0059x over previous
import functools

import jax
import jax.numpy as jnp
from jax import lax
from jax.experimental import pallas as pl
from jax.experimental.pallas import tpu as pltpu
from jax.experimental.pallas import tpu_sc as plsc

BATCH = 4096
SEQ = 200
DIMS = 64
LANES = 16
NW = 32
B_PER_W = BATCH // NW
SB = 2
NBUF = 2
NBLOCKS = SEQ // SB
ROWS_PER_BLOCK = SB * B_PER_W


def _suffix_embed_kernel(x_hbm, pe_hbm, out_hbm, x_v, idx_v, pe_v,
                         buf0, buf1, wsem0, wsem1):
    wid = lax.axis_index("s") * 2 + lax.axis_index("c")
    b_base = wid * B_PER_W

    pltpu.sync_copy(x_hbm.at[pl.ds(b_base * SEQ, B_PER_W * SEQ)], x_v)
    pltpu.sync_copy(pe_hbm, pe_v)

    iota = lax.iota(jnp.int32, LANES)
    one = jnp.ones((LANES,), jnp.int32)
    zero = jnp.zeros((LANES,), jnp.int32)

    row_bases = [(g * LANES + iota) * SEQ for g in range(B_PER_W // LANES)]

    def scan_body(i, accs):
        s = SEQ - 1 - i
        out_accs = []
        for g, acc in enumerate(accs):
            col = plsc.load_gather(x_v, [row_bases[g] + s])
            acc = acc + jnp.where(col != 0, one, zero)
            idx_v[pl.ds(s * B_PER_W + g * LANES, LANES)] = acc
            out_accs.append(acc)
        return tuple(out_accs)

    lax.fori_loop(0, SEQ, scan_body, tuple(zero for _ in range(B_PER_W // LANES)))

    bufs = (buf0, buf1)
    wsems = (wsem0, wsem1)
    d_iotas = [iota + k * LANES for k in range(DIMS // LANES)]

    def write_block(i, b, wait):
        for k in range(SB):
            cp = pltpu.make_async_copy(
                bufs[b].at[pl.ds(k * B_PER_W, B_PER_W), :],
                out_hbm.at[pl.ds((i * SB + k) * BATCH + b_base, B_PER_W), :],
                wsems[b],
            )
            if wait:
                cp.wait()
            else:
                cp.start()

    def fill_block(i, b):
        buf = bufs[b]

        def grp_body(g16, carry, buf=buf, i=i):
            base = g16 * LANES
            T = idx_v[pl.ds(i * ROWS_PER_BLOCK + base, LANES)] * DIMS
            for q in range(LANES // 4):
                gs = []
                for l in range(q * 4, q * 4 + 4):
                    t64 = T[l]
                    gs.append([plsc.load_gather(pe_v, [t64 + d_iotas[k]])
                               for k in range(DIMS // LANES)])
                for li, l in enumerate(range(q * 4, q * 4 + 4)):
                    for k in range(DIMS // LANES):
                        buf[base + l, pl.ds(k * LANES, LANES)] = gs[li][k]
            return carry

        lax.fori_loop(0, ROWS_PER_BLOCK // LANES, grp_body, 0)

    def outer(o, carry):
        j = o * NBUF
        for b in range(NBUF):
            i = j + b

            @pl.when(o > 0)
            def _(b=b, i=i):
                write_block(i - NBUF, b, wait=True)

            fill_block(i, b)
            write_block(i, b, wait=False)
        return carry

    lax.fori_loop(0, NBLOCKS // NBUF, outer, 0)

    for b in range(NBUF):
        write_block(NBLOCKS - NBUF + b, b, wait=True)


@jax.jit
def kernel(x, pe):
    mesh = plsc.VectorSubcoreMesh(core_axis_name="c", subcore_axis_name="s")
    fn = functools.partial(
        pl.kernel,
        mesh=mesh,
        compiler_params=pltpu.CompilerParams(
            needs_layout_passes=False, use_tc_tiling_on_sc=False
        ),
        out_type=jax.ShapeDtypeStruct((SEQ * BATCH, DIMS), jnp.float32),
        scratch_types=[
            pltpu.VMEM((B_PER_W * SEQ,), jnp.int32),
            pltpu.VMEM((SEQ * B_PER_W,), jnp.int32),
            pltpu.VMEM((256 * DIMS,), jnp.float32),
            pltpu.VMEM((ROWS_PER_BLOCK, DIMS), jnp.float32),
            pltpu.VMEM((ROWS_PER_BLOCK, DIMS), jnp.float32),
            pltpu.SemaphoreType.DMA,
            pltpu.SemaphoreType.DMA,
        ],
    )(_suffix_embed_kernel)
    return fn(x.reshape(-1), pe.reshape(-1)).reshape(SEQ, BATCH, DIMS)
